# Initial kernel scaffold; baseline (speedup 1.0000x reference)
import jax
import jax.numpy as jnp
from jax.experimental import pallas as pl


def kernel(group_by_key, stacked_embeddings):
    raise NotImplementedError("write your pallas kernel here")


# speedup vs baseline: 6.8024x; 6.8024x over previous
import functools

import jax
import jax.numpy as jnp
from jax import lax
from jax.experimental import pallas as pl
from jax.experimental.pallas import tpu as pltpu
from jax.experimental.pallas import tpu_sc as plsc

N = 32768
D = 128
K = 1024
NC = 2
NS = 16
NW = NC * NS
ROWS_PER_W = N // NW
CHUNK = 128
NCHUNKS = ROWS_PER_W // CHUNK
GPT = K // NS

_mesh = plsc.VectorSubcoreMesh(
    core_axis_name="c", subcore_axis_name="s", num_cores=NC, num_subcores=NS)

_f32 = jnp.float32


def _fill(ref, rows, width, val):
    v = jnp.full((16,), val, _f32)

    def row(r, _):
        for cc in range(width // 16):
            ref[r, pl.ds(cc * 16, 16)] = v
        return 0

    lax.fori_loop(0, rows, row, 0)


@functools.partial(
    pl.kernel,
    out_type=(
        jax.ShapeDtypeStruct((NC, K, D), _f32),
        jax.ShapeDtypeStruct((NC, K, D), _f32),
        jax.ShapeDtypeStruct((NC, K, D), _f32),
    ),
    mesh=_mesh,
    scratch_types=(
        pltpu.VMEM((CHUNK,), jnp.int32),
        pltpu.VMEM((CHUNK, D), _f32),
        pltpu.VMEM((CHUNK, D), _f32),
        pltpu.VMEM((GPT, D), _f32),
        pltpu.VMEM_SHARED((K, D), _f32),
        pltpu.VMEM_SHARED((K, D), _f32),
        pltpu.VMEM_SHARED((K, D), _f32),
    ),
)
def _accum(key_hbm, x_hbm, s_out, q_out, c_out,
           idx_v, xbuf, ones_v, zbuf, s_sh, q_sh, c_sh):
    c = lax.axis_index("c")
    s = lax.axis_index("s")
    wid = c * NS + s

    _fill(zbuf, GPT, D, 0.0)
    _fill(ones_v, CHUNK, D, 1.0)
    pltpu.sync_copy(zbuf, s_sh.at[pl.ds(s * GPT, GPT)])
    pltpu.sync_copy(zbuf, q_sh.at[pl.ds(s * GPT, GPT)])
    pltpu.sync_copy(zbuf, c_sh.at[pl.ds(s * GPT, GPT)])
    plsc.subcore_barrier()

    base = wid * ROWS_PER_W

    def chunk(j, _):
        rb = base + j * CHUNK
        pltpu.sync_copy(key_hbm.at[pl.ds(rb, CHUNK)], idx_v)
        pltpu.sync_copy(x_hbm.at[pl.ds(rb, CHUNK)], xbuf)
        pltpu.sync_copy(xbuf, s_sh.at[idx_v], add=True)
        pltpu.sync_copy(ones_v, c_sh.at[idx_v], add=True)

        def srow(r, _):
            for cc in range(D // 16):
                v = xbuf[r, pl.ds(cc * 16, 16)]
                xbuf[r, pl.ds(cc * 16, 16)] = v * v
            return 0

        lax.fori_loop(0, CHUNK, srow, 0)
        pltpu.sync_copy(xbuf, q_sh.at[idx_v], add=True)
        return 0

    lax.fori_loop(0, NCHUNKS, chunk, 0)
    plsc.subcore_barrier()

    gb = s * GPT
    pltpu.sync_copy(s_sh.at[pl.ds(gb, GPT)], zbuf)
    pltpu.sync_copy(zbuf, s_out.at[c, pl.ds(gb, GPT)])
    pltpu.sync_copy(q_sh.at[pl.ds(gb, GPT)], zbuf)
    pltpu.sync_copy(zbuf, q_out.at[c, pl.ds(gb, GPT)])
    pltpu.sync_copy(c_sh.at[pl.ds(gb, GPT)], zbuf)
    pltpu.sync_copy(zbuf, c_out.at[c, pl.ds(gb, GPT)])


def _finalize_body(s2_ref, q2_ref, c2_ref, mean_ref, var_ref):
    s = s2_ref[0] + s2_ref[1]
    q = q2_ref[0] + q2_ref[1]
    cnt = c2_ref[0, :, 0:1] + c2_ref[1, :, 0:1]
    rc = 1.0 / jnp.maximum(cnt, 1.0)
    mean = s * rc
    var = q * rc - mean * mean
    mean_ref[...] = mean
    var_ref[...] = var


_finalize = pl.pallas_call(
    _finalize_body,
    out_shape=(
        jax.ShapeDtypeStruct((K, D), _f32),
        jax.ShapeDtypeStruct((K, D), _f32),
    ),
)


@functools.partial(
    pl.kernel,
    out_type=(
        jax.ShapeDtypeStruct((N, D), _f32),
        jax.ShapeDtypeStruct((N, D), _f32),
    ),
    mesh=_mesh,
    scratch_types=(
        pltpu.VMEM((CHUNK,), jnp.int32),
        pltpu.VMEM((CHUNK, D), _f32),
        pltpu.VMEM((CHUNK, D), _f32),
        pltpu.SemaphoreType.DMA,
        pltpu.SemaphoreType.DMA,
    ),
)
def _gather(key_hbm, mtab, vtab, om, ov, idx_v, bm, bv, sem_m, sem_v):
    c = lax.axis_index("c")
    s = lax.axis_index("s")
    wid = c * NS + s
    base = wid * ROWS_PER_W

    def chunk(j, _):
        rb = base + j * CHUNK
        pltpu.sync_copy(key_hbm.at[pl.ds(rb, CHUNK)], idx_v)
        cm = pltpu.async_copy(mtab.at[idx_v], bm, sem_m)
        cv = pltpu.async_copy(vtab.at[idx_v], bv, sem_v)
        cm.wait()
        cv.wait()
        pltpu.sync_copy(bm, om.at[pl.ds(rb, CHUNK)])
        pltpu.sync_copy(bv, ov.at[pl.ds(rb, CHUNK)])
        return 0

    lax.fori_loop(0, NCHUNKS, chunk, 0)


def kernel(group_by_key, stacked_embeddings):
    key = group_by_key.astype(jnp.int32)
    x = stacked_embeddings
    s2, q2, c2 = _accum(key, x)
    mean_tab, var_tab = _finalize(s2, q2, c2)
    return _gather(key, mean_tab, var_tab)


# speedup vs baseline: 7.0376x; 1.0346x over previous
import functools

import jax
import jax.numpy as jnp
from jax import lax
from jax.experimental import pallas as pl
from jax.experimental.pallas import tpu as pltpu
from jax.experimental.pallas import tpu_sc as plsc

N = 32768
D = 128
K = 1024
NC = 2
NS = 16
NW = NC * NS
ROWS_PER_W = N // NW
CHUNK = 128
NCHUNKS = ROWS_PER_W // CHUNK
GPT = K // NS

_mesh = plsc.VectorSubcoreMesh(
    core_axis_name="c", subcore_axis_name="s", num_cores=NC, num_subcores=NS)

_f32 = jnp.float32


def _fill(ref, rows, width, val):
    v = jnp.full((16,), val, _f32)

    def row(r, _):
        for cc in range(width // 16):
            ref[r, pl.ds(cc * 16, 16)] = v
        return 0

    lax.fori_loop(0, rows, row, 0)


@functools.partial(
    pl.kernel,
    out_type=(
        jax.ShapeDtypeStruct((NC, K, D), _f32),
        jax.ShapeDtypeStruct((NC, K, D), _f32),
        jax.ShapeDtypeStruct((NC, K, D), _f32),
    ),
    mesh=_mesh,
    scratch_types=(
        pltpu.VMEM((CHUNK,), jnp.int32),
        pltpu.VMEM((CHUNK, D), _f32),
        pltpu.VMEM((CHUNK, D), _f32),
        pltpu.VMEM((GPT, D), _f32),
        pltpu.VMEM_SHARED((K, D), _f32),
        pltpu.VMEM_SHARED((K, D), _f32),
        pltpu.VMEM_SHARED((K, D), _f32),
    ),
)
def _accum(key_hbm, x_hbm, s_out, q_out, c_out,
           idx_v, xbuf, ones_v, zbuf, s_sh, q_sh, c_sh):
    c = lax.axis_index("c")
    s = lax.axis_index("s")
    wid = c * NS + s

    _fill(zbuf, GPT, D, 0.0)
    _fill(ones_v, CHUNK, D, 1.0)
    pltpu.sync_copy(zbuf, s_sh.at[pl.ds(s * GPT, GPT)])
    pltpu.sync_copy(zbuf, q_sh.at[pl.ds(s * GPT, GPT)])
    pltpu.sync_copy(zbuf, c_sh.at[pl.ds(s * GPT, GPT)])
    plsc.subcore_barrier()

    base = wid * ROWS_PER_W

    def chunk(j, _):
        rb = base + j * CHUNK
        pltpu.sync_copy(key_hbm.at[pl.ds(rb, CHUNK)], idx_v)
        pltpu.sync_copy(x_hbm.at[pl.ds(rb, CHUNK)], xbuf)
        pltpu.sync_copy(xbuf, s_sh.at[idx_v], add=True)
        pltpu.sync_copy(ones_v, c_sh.at[idx_v], add=True)

        def srow(r, _):
            for cc in range(D // 16):
                v = xbuf[r, pl.ds(cc * 16, 16)]
                xbuf[r, pl.ds(cc * 16, 16)] = v * v
            return 0

        lax.fori_loop(0, CHUNK, srow, 0)
        pltpu.sync_copy(xbuf, q_sh.at[idx_v], add=True)
        return 0

    lax.fori_loop(0, NCHUNKS, chunk, 0)
    plsc.subcore_barrier()

    gb = s * GPT
    pltpu.sync_copy(s_sh.at[pl.ds(gb, GPT)], zbuf)
    pltpu.sync_copy(zbuf, s_out.at[c, pl.ds(gb, GPT)])
    pltpu.sync_copy(q_sh.at[pl.ds(gb, GPT)], zbuf)
    pltpu.sync_copy(zbuf, q_out.at[c, pl.ds(gb, GPT)])
    pltpu.sync_copy(c_sh.at[pl.ds(gb, GPT)], zbuf)
    pltpu.sync_copy(zbuf, c_out.at[c, pl.ds(gb, GPT)])


@functools.partial(
    pl.kernel,
    out_type=(
        jax.ShapeDtypeStruct((N, D), _f32),
        jax.ShapeDtypeStruct((N, D), _f32),
    ),
    mesh=_mesh,
    scratch_types=(
        pltpu.VMEM((CHUNK,), jnp.int32),
        pltpu.VMEM((CHUNK, D), _f32),
        pltpu.VMEM((CHUNK, D), _f32),
        pltpu.VMEM((GPT, D), _f32),
        pltpu.VMEM((GPT, D), _f32),
        pltpu.VMEM((GPT, D), _f32),
        pltpu.VMEM((GPT, D), _f32),
        pltpu.VMEM_SHARED((K, D), _f32),
        pltpu.VMEM_SHARED((K, D), _f32),
        pltpu.SemaphoreType.DMA,
        pltpu.SemaphoreType.DMA,
    ),
)
def _finalize_gather(key_hbm, s2, q2, c2, om, ov,
                     idx_v, bm, bv, t_s, t_q, t_r, t_t, m_sh, v_sh,
                     sem_m, sem_v):
    c = lax.axis_index("c")
    s = lax.axis_index("s")
    wid = c * NS + s
    gb = s * GPT

    pltpu.sync_copy(c2.at[0, pl.ds(gb, GPT)], t_r)
    pltpu.sync_copy(c2.at[1, pl.ds(gb, GPT)], t_t)
    one = jnp.ones((16,), _f32)

    def rrow(r, _):
        for cc in range(D // 16):
            sl = pl.ds(cc * 16, 16)
            t_r[r, sl] = one / jnp.maximum(t_r[r, sl] + t_t[r, sl], one)
        return 0

    lax.fori_loop(0, GPT, rrow, 0)

    pltpu.sync_copy(s2.at[0, pl.ds(gb, GPT)], t_s)
    pltpu.sync_copy(s2.at[1, pl.ds(gb, GPT)], t_t)

    def mrow(r, _):
        for cc in range(D // 16):
            sl = pl.ds(cc * 16, 16)
            t_s[r, sl] = (t_s[r, sl] + t_t[r, sl]) * t_r[r, sl]
        return 0

    lax.fori_loop(0, GPT, mrow, 0)

    pltpu.sync_copy(q2.at[0, pl.ds(gb, GPT)], t_q)
    pltpu.sync_copy(q2.at[1, pl.ds(gb, GPT)], t_t)

    def vrow(r, _):
        for cc in range(D // 16):
            sl = pl.ds(cc * 16, 16)
            m = t_s[r, sl]
            t_q[r, sl] = (t_q[r, sl] + t_t[r, sl]) * t_r[r, sl] - m * m
        return 0

    lax.fori_loop(0, GPT, vrow, 0)

    pltpu.sync_copy(t_s, m_sh.at[pl.ds(gb, GPT)])
    pltpu.sync_copy(t_q, v_sh.at[pl.ds(gb, GPT)])
    plsc.subcore_barrier()

    base = wid * ROWS_PER_W

    def chunk(j, _):
        rb = base + j * CHUNK
        pltpu.sync_copy(key_hbm.at[pl.ds(rb, CHUNK)], idx_v)
        cm = pltpu.async_copy(m_sh.at[idx_v], bm, sem_m)
        cv = pltpu.async_copy(v_sh.at[idx_v], bv, sem_v)
        cm.wait()
        cv.wait()
        pltpu.sync_copy(bm, om.at[pl.ds(rb, CHUNK)])
        pltpu.sync_copy(bv, ov.at[pl.ds(rb, CHUNK)])
        return 0

    lax.fori_loop(0, NCHUNKS, chunk, 0)


def kernel(group_by_key, stacked_embeddings):
    key = group_by_key.astype(jnp.int32)
    x = stacked_embeddings
    s2, q2, c2 = _accum(key, x)
    return _finalize_gather(key, s2, q2, c2)


# speedup vs baseline: 8.7724x; 1.2465x over previous
import functools

import jax
import jax.numpy as jnp
from jax import lax
from jax.experimental import pallas as pl
from jax.experimental.pallas import tpu as pltpu
from jax.experimental.pallas import tpu_sc as plsc

N = 32768
D = 128
K = 1024
NC = 2
NS = 16
NW = NC * NS
ROWS_PER_W = N // NW
CHUNK = 128
NCHUNKS = ROWS_PER_W // CHUNK
GPT = K // NS

_mesh = plsc.VectorSubcoreMesh(
    core_axis_name="c", subcore_axis_name="s", num_cores=NC, num_subcores=NS)

_f32 = jnp.float32


def _fill(ref, rows, width, val):
    v = jnp.full((16,), val, _f32)

    def row(r, _):
        for cc in range(width // 16):
            ref[r, pl.ds(cc * 16, 16)] = v
        return 0

    lax.fori_loop(0, rows, row, 0)


@functools.partial(
    pl.kernel,
    out_type=(
        jax.ShapeDtypeStruct((NC, K, D), _f32),
        jax.ShapeDtypeStruct((NC, K, D), _f32),
        jax.ShapeDtypeStruct((NC, K, D), _f32),
    ),
    mesh=_mesh,
    scratch_types=(
        pltpu.VMEM((CHUNK,), jnp.int32),
        pltpu.VMEM((CHUNK,), jnp.int32),
        pltpu.VMEM((CHUNK, D), _f32),
        pltpu.VMEM((CHUNK, D), _f32),
        pltpu.VMEM((CHUNK, D), _f32),
        pltpu.VMEM((CHUNK, D), _f32),
        pltpu.VMEM((CHUNK, D), _f32),
        pltpu.VMEM((GPT, D), _f32),
        pltpu.VMEM_SHARED((K, D), _f32),
        pltpu.VMEM_SHARED((K, D), _f32),
        pltpu.VMEM_SHARED((K, D), _f32),
        pltpu.SemaphoreType.DMA,
        pltpu.SemaphoreType.DMA,
        pltpu.SemaphoreType.DMA,
        pltpu.SemaphoreType.DMA,
        pltpu.SemaphoreType.DMA,
        pltpu.SemaphoreType.DMA,
        pltpu.SemaphoreType.DMA,
        pltpu.SemaphoreType.DMA,
        pltpu.SemaphoreType.DMA,
        pltpu.SemaphoreType.DMA,
    ),
)
def _accum(key_hbm, x_hbm, s_out, q_out, c_out,
           idx0, idx1, x0, x1, sq0, sq1, ones_v, zbuf, s_sh, q_sh, c_sh,
           sem_li0, sem_li1, sem_lx0, sem_lx1, sem_sx0, sem_sx1,
           sem_sq0, sem_sq1, sem_c0, sem_c1):
    c = lax.axis_index("c")
    s = lax.axis_index("s")
    wid = c * NS + s
    base = wid * ROWS_PER_W

    idx = (idx0, idx1)
    xb = (x0, x1)
    sqb = (sq0, sq1)
    sem_li = (sem_li0, sem_li1)
    sem_lx = (sem_lx0, sem_lx1)
    sem_sx = (sem_sx0, sem_sx1)
    sem_sq = (sem_sq0, sem_sq1)
    sem_c = (sem_c0, sem_c1)

    ld_i = [None] * NCHUNKS
    ld_x = [None] * NCHUNKS
    sc_x = [None] * NCHUNKS
    sc_q = [None] * NCHUNKS
    sc_c = [None] * NCHUNKS

    def issue_load(j):
        p = j % 2
        rb = base + j * CHUNK
        ld_i[j] = pltpu.async_copy(key_hbm.at[pl.ds(rb, CHUNK)], idx[p], sem_li[p])
        ld_x[j] = pltpu.async_copy(x_hbm.at[pl.ds(rb, CHUNK)], xb[p], sem_lx[p])

    issue_load(0)

    _fill(zbuf, GPT, D, 0.0)
    _fill(ones_v, CHUNK, D, 1.0)
    gb = s * GPT
    pltpu.sync_copy(zbuf, s_sh.at[pl.ds(gb, GPT)])
    pltpu.sync_copy(zbuf, q_sh.at[pl.ds(gb, GPT)])
    pltpu.sync_copy(zbuf, c_sh.at[pl.ds(gb, GPT)])
    plsc.subcore_barrier()

    for j in range(NCHUNKS):
        p = j % 2
        ld_i[j].wait()
        ld_x[j].wait()
        sc_x[j] = pltpu.async_copy(xb[p], s_sh.at[idx[p]], sem_sx[p], add=True)
        sc_c[j] = pltpu.async_copy(ones_v, c_sh.at[idx[p]], sem_c[p], add=True)
        if j + 1 < NCHUNKS:
            if j >= 1:
                sc_x[j - 1].wait()
                sc_c[j - 1].wait()
                sc_q[j - 1].wait()
            issue_load(j + 1)

        def srow(r, _):
            for cc in range(D // 16):
                sl = pl.ds(cc * 16, 16)
                v = xb[p][r, sl]
                sqb[p][r, sl] = v * v
            return 0

        lax.fori_loop(0, CHUNK, srow, 0)
        sc_q[j] = pltpu.async_copy(sqb[p], q_sh.at[idx[p]], sem_sq[p], add=True)

    for j in (NCHUNKS - 2, NCHUNKS - 1):
        sc_x[j].wait()
        sc_c[j].wait()
        sc_q[j].wait()
    plsc.subcore_barrier()

    pltpu.sync_copy(s_sh.at[pl.ds(gb, GPT)], zbuf)
    pltpu.sync_copy(zbuf, s_out.at[c, pl.ds(gb, GPT)])
    pltpu.sync_copy(q_sh.at[pl.ds(gb, GPT)], zbuf)
    pltpu.sync_copy(zbuf, q_out.at[c, pl.ds(gb, GPT)])
    pltpu.sync_copy(c_sh.at[pl.ds(gb, GPT)], zbuf)
    pltpu.sync_copy(zbuf, c_out.at[c, pl.ds(gb, GPT)])


@functools.partial(
    pl.kernel,
    out_type=(
        jax.ShapeDtypeStruct((N, D), _f32),
        jax.ShapeDtypeStruct((N, D), _f32),
    ),
    mesh=_mesh,
    scratch_types=(
        pltpu.VMEM((CHUNK,), jnp.int32),
        pltpu.VMEM((CHUNK,), jnp.int32),
        pltpu.VMEM((CHUNK, D), _f32),
        pltpu.VMEM((CHUNK, D), _f32),
        pltpu.VMEM((CHUNK, D), _f32),
        pltpu.VMEM((CHUNK, D), _f32),
        pltpu.VMEM((GPT, D), _f32),
        pltpu.VMEM((GPT, D), _f32),
        pltpu.VMEM((GPT, D), _f32),
        pltpu.VMEM((GPT, D), _f32),
        pltpu.VMEM_SHARED((K, D), _f32),
        pltpu.VMEM_SHARED((K, D), _f32),
        pltpu.SemaphoreType.DMA,
        pltpu.SemaphoreType.DMA,
        pltpu.SemaphoreType.DMA,
        pltpu.SemaphoreType.DMA,
        pltpu.SemaphoreType.DMA,
        pltpu.SemaphoreType.DMA,
        pltpu.SemaphoreType.DMA,
        pltpu.SemaphoreType.DMA,
        pltpu.SemaphoreType.DMA,
        pltpu.SemaphoreType.DMA,
    ),
)
def _finalize_gather(key_hbm, s2, q2, c2, om, ov,
                     idxg0, idxg1, bm0, bm1, bv0, bv1, t_s, t_q, t_r, t_t,
                     m_sh, v_sh,
                     sem_li0, sem_li1, sem_gm0, sem_gm1, sem_gv0, sem_gv1,
                     sem_wm0, sem_wm1, sem_wv0, sem_wv1):
    c = lax.axis_index("c")
    s = lax.axis_index("s")
    wid = c * NS + s
    gb = s * GPT
    base = wid * ROWS_PER_W

    idxg = (idxg0, idxg1)
    bm = (bm0, bm1)
    bv = (bv0, bv1)
    sem_li = (sem_li0, sem_li1)
    sem_gm = (sem_gm0, sem_gm1)
    sem_gv = (sem_gv0, sem_gv1)
    sem_wm = (sem_wm0, sem_wm1)
    sem_wv = (sem_wv0, sem_wv1)

    ld_i = [None] * NCHUNKS
    ld_i[0] = pltpu.async_copy(
        key_hbm.at[pl.ds(base, CHUNK)], idxg[0], sem_li[0])

    pltpu.sync_copy(c2.at[0, pl.ds(gb, GPT)], t_r)
    pltpu.sync_copy(c2.at[1, pl.ds(gb, GPT)], t_t)
    one = jnp.ones((16,), _f32)

    def rrow(r, _):
        for cc in range(D // 16):
            sl = pl.ds(cc * 16, 16)
            t_r[r, sl] = one / jnp.maximum(t_r[r, sl] + t_t[r, sl], one)
        return 0

    lax.fori_loop(0, GPT, rrow, 0)

    pltpu.sync_copy(s2.at[0, pl.ds(gb, GPT)], t_s)
    pltpu.sync_copy(s2.at[1, pl.ds(gb, GPT)], t_t)

    def mrow(r, _):
        for cc in range(D // 16):
            sl = pl.ds(cc * 16, 16)
            t_s[r, sl] = (t_s[r, sl] + t_t[r, sl]) * t_r[r, sl]
        return 0

    lax.fori_loop(0, GPT, mrow, 0)

    pltpu.sync_copy(q2.at[0, pl.ds(gb, GPT)], t_q)
    pltpu.sync_copy(q2.at[1, pl.ds(gb, GPT)], t_t)

    def vrow(r, _):
        for cc in range(D // 16):
            sl = pl.ds(cc * 16, 16)
            m = t_s[r, sl]
            t_q[r, sl] = (t_q[r, sl] + t_t[r, sl]) * t_r[r, sl] - m * m
        return 0

    lax.fori_loop(0, GPT, vrow, 0)

    pltpu.sync_copy(t_s, m_sh.at[pl.ds(gb, GPT)])
    pltpu.sync_copy(t_q, v_sh.at[pl.ds(gb, GPT)])
    plsc.subcore_barrier()

    g_m = [None] * NCHUNKS
    g_v = [None] * NCHUNKS
    w_m = [None] * NCHUNKS
    w_v = [None] * NCHUNKS
    for j in range(NCHUNKS):
        p = j % 2
        rb = base + j * CHUNK
        ld_i[j].wait()
        if j >= 2:
            w_m[j - 2].wait()
            w_v[j - 2].wait()
        g_m[j] = pltpu.async_copy(m_sh.at[idxg[p]], bm[p], sem_gm[p])
        g_v[j] = pltpu.async_copy(v_sh.at[idxg[p]], bv[p], sem_gv[p])
        g_m[j].wait()
        g_v[j].wait()
        w_m[j] = pltpu.async_copy(bm[p], om.at[pl.ds(rb, CHUNK)], sem_wm[p])
        w_v[j] = pltpu.async_copy(bv[p], ov.at[pl.ds(rb, CHUNK)], sem_wv[p])
        if j + 1 < NCHUNKS:
            q = (j + 1) % 2
            ld_i[j + 1] = pltpu.async_copy(
                key_hbm.at[pl.ds(base + (j + 1) * CHUNK, CHUNK)],
                idxg[q], sem_li[q])
    for j in (NCHUNKS - 2, NCHUNKS - 1):
        w_m[j].wait()
        w_v[j].wait()


def kernel(group_by_key, stacked_embeddings):
    key = group_by_key.astype(jnp.int32)
    x = stacked_embeddings
    s2, q2, c2 = _accum(key, x)
    return _finalize_gather(key, s2, q2, c2)
